# baseline (speedup 1.0000x reference)
import functools

import jax
import jax.numpy as jnp
from jax import lax
from jax.experimental import pallas as pl
from jax.experimental.pallas import tpu as pltpu
from jax.experimental.pallas import tpu_sc as plsc

N = 10000
E = 320000
G = 64
F_IN = 128
H = 32
N_MICRO = 30

NC = 2
NS = 16
NW = NC * NS
CH = 128
NCH = 80
NBUF = 4
EP = NW * NCH * CH
NP = 10112
RPT = NP // NS

_mesh = plsc.VectorSubcoreMesh(
    core_axis_name="c", subcore_axis_name="s", num_cores=NC, num_subcores=NS)
_sc_params = pltpu.CompilerParams(use_tc_tiling_on_sc=False)


@functools.partial(
    pl.kernel,
    out_type=jax.ShapeDtypeStruct((NC, NP, 16), jnp.float32),
    mesh=_mesh,
    scratch_types=[
        pltpu.VMEM((NCH, CH), jnp.int32),
        pltpu.VMEM((CH, 16), jnp.float32),
        pltpu.VMEM_SHARED((NP, 16), jnp.float32),
    ],
    compiler_params=_sc_params,
)
def _sc_degree(dst_hbm, ones_hbm, z16_hbm, out_hbm, dst_v, ones_v, acc_sh):
    c = lax.axis_index("c")
    s = lax.axis_index("s")
    wid = c * NS + s
    pltpu.sync_copy(z16_hbm.at[pl.ds(s * RPT, RPT)], acc_sh.at[pl.ds(s * RPT, RPT)])
    pltpu.sync_copy(dst_hbm.at[wid], dst_v)
    pltpu.sync_copy(ones_hbm, ones_v)
    plsc.subcore_barrier()

    def body(j, carry):
        pltpu.sync_copy(ones_v, acc_sh.at[dst_v.at[j]], add=True)
        return carry

    lax.fori_loop(0, NCH, body, 0)
    plsc.subcore_barrier()
    pltpu.sync_copy(acc_sh.at[pl.ds(s * RPT, RPT)],
                    out_hbm.at[c, pl.ds(s * RPT, RPT)])


@functools.partial(
    pl.kernel,
    out_type=jax.ShapeDtypeStruct((NC, NP, H), jnp.float32),
    mesh=_mesh,
    scratch_types=[
        pltpu.VMEM((NCH, CH), jnp.int32),
        pltpu.VMEM((NCH, CH), jnp.int32),
        [pltpu.VMEM((CH, H), jnp.float32)] * NBUF,
        [pltpu.SemaphoreType.DMA] * NBUF,
        [pltpu.SemaphoreType.DMA] * NBUF,
        pltpu.VMEM_SHARED((NP, H), jnp.float32),
    ],
    compiler_params=_sc_params,
)
def _sc_agg(hp_hbm, src_hbm, dst_hbm, z_hbm, out_hbm, src_v, dst_v, rows,
            gsem, ssem, acc_sh):
    c = lax.axis_index("c")
    s = lax.axis_index("s")
    wid = c * NS + s
    pltpu.sync_copy(z_hbm.at[pl.ds(s * RPT, RPT)], acc_sh.at[pl.ds(s * RPT, RPT)])
    pltpu.sync_copy(src_hbm.at[wid], src_v)
    pltpu.sync_copy(dst_hbm.at[wid], dst_v)
    plsc.subcore_barrier()

    def start_gather(i, chunk):
        pltpu.async_copy(hp_hbm.at[src_v.at[chunk]], rows[i], gsem[i])

    def wait_gather(i, chunk):
        pltpu.make_async_copy(hp_hbm.at[src_v.at[chunk]], rows[i], gsem[i]).wait()

    def start_scat(i, chunk):
        pltpu.async_copy(rows[i], acc_sh.at[dst_v.at[chunk]], ssem[i], add=True)

    def wait_scat(i, chunk):
        pltpu.make_async_copy(rows[i], acc_sh.at[dst_v.at[chunk]], ssem[i]).wait()

    for i in range(NBUF):
        start_gather(i, i)

    def body(j, carry):
        base = j * NBUF
        for i in range(NBUF):
            wait_gather(i, base + i)
            start_scat(i, base + i)
        for i in range(NBUF):
            wait_scat(i, base + i)
            start_gather(i, base + NBUF + i)
        return carry

    lax.fori_loop(0, NCH // NBUF - 1, body, 0)
    base = NCH - NBUF
    for i in range(NBUF):
        wait_gather(i, base + i)
        start_scat(i, base + i)
    for i in range(NBUF):
        wait_scat(i, base + i)
    plsc.subcore_barrier()
    pltpu.sync_copy(acc_sh.at[pl.ds(s * RPT, RPT)],
                    out_hbm.at[c, pl.ds(s * RPT, RPT)])


def _tc_pre_body(x_ref, w1_ref, degp_ref, hp_ref, dinv_ref):
    deg = degp_ref[0, :, 0:1] + degp_ref[1, :, 0:1] + 1.0
    dinv = lax.rsqrt(deg)
    h = jnp.dot(x_ref[...], w1_ref[...], preferred_element_type=jnp.float32)
    hp_ref[...] = dinv * h
    dinv_ref[...] = dinv


def _tc_mid_body(aggp_ref, hp_ref, dinv_ref, b_ref, w_ref, out_ref):
    dinv = dinv_ref[...]
    pre = dinv * (aggp_ref[0] + aggp_ref[1] + hp_ref[...]) + b_ref[...]
    a = jnp.maximum(pre, 0.0)
    out_ref[...] = dinv * jnp.dot(a, w_ref[...], preferred_element_type=jnp.float32)


def _tc_head_body(aggp_ref, hp_ref, dinv_ref, b_ref, batch_ref,
                  fw1_ref, fb1_ref, fw2_ref, fb2_ref, out_ref):
    dinv = dinv_ref[...]
    pre = dinv * (aggp_ref[0] + aggp_ref[1] + hp_ref[...]) + b_ref[...]
    a = jnp.maximum(pre, 0.0)[:N, :]
    gid = lax.broadcasted_iota(jnp.int32, (G, N), 0)
    oh = (gid == batch_ref[...]).astype(jnp.float32)
    sums = jnp.dot(oh, a, preferred_element_type=jnp.float32)
    cnt = jnp.sum(oh, axis=1, keepdims=True)
    pooled = sums / jnp.maximum(cnt, 1.0)
    z = jnp.maximum(
        jnp.dot(pooled, fw1_ref[...], preferred_element_type=jnp.float32)
        + fb1_ref[...], 0.0)
    z2 = jnp.dot(z, fw2_ref[...], preferred_element_type=jnp.float32) + fb2_ref[...]
    m = jnp.max(z2, axis=1, keepdims=True)
    e = jnp.exp(z2 - m)
    out_ref[...] = e / jnp.sum(e, axis=1, keepdims=True)


_tc_pre = pl.pallas_call(
    _tc_pre_body,
    out_shape=(jax.ShapeDtypeStruct((NP, H), jnp.float32),
               jax.ShapeDtypeStruct((NP, 1), jnp.float32)),
)

_tc_mid = pl.pallas_call(
    _tc_mid_body,
    out_shape=jax.ShapeDtypeStruct((NP, H), jnp.float32),
)

_tc_head = pl.pallas_call(
    _tc_head_body,
    out_shape=jax.ShapeDtypeStruct((G, N_MICRO), jnp.float32),
)


def kernel(x, edge_index, batch, W1, b1, W2, b2, W3, b3, W4, b4,
           fW1, fb1, fW2, fb2):
    src = edge_index[0]
    dst = edge_index[1]
    pad = EP - E
    src3 = jnp.concatenate([src, jnp.zeros((pad,), jnp.int32)]).reshape(NW, NCH, CH)
    dst3 = jnp.concatenate([dst, jnp.full((pad,), N, jnp.int32)]).reshape(NW, NCH, CH)

    ones16 = jnp.ones((CH, 16), jnp.float32)
    z16 = jnp.zeros((NP, 16), jnp.float32)
    zH = jnp.zeros((NP, H), jnp.float32)
    x_pad = jnp.concatenate([x, jnp.zeros((NP - N, F_IN), x.dtype)], axis=0)
    batch2 = batch.reshape(1, N)

    degp = _sc_degree(dst3, ones16, z16)
    hp, dinv = _tc_pre(x_pad, W1, degp)

    aggp = _sc_agg(hp, src3, dst3, zH)
    hp = _tc_mid(aggp, hp, dinv, b1.reshape(1, H), W2)

    aggp = _sc_agg(hp, src3, dst3, zH)
    hp = _tc_mid(aggp, hp, dinv, b2.reshape(1, H), W3)

    aggp = _sc_agg(hp, src3, dst3, zH)
    hp = _tc_mid(aggp, hp, dinv, b3.reshape(1, H), W4)

    aggp = _sc_agg(hp, src3, dst3, zH)
    return _tc_head(aggp, hp, dinv, b4.reshape(1, H), batch2,
                    fW1, fb1.reshape(1, 64), fW2, fb2.reshape(1, N_MICRO))


# speedup vs baseline: 1.0253x; 1.0253x over previous
import functools

import jax
import jax.numpy as jnp
from jax import lax
from jax.experimental import pallas as pl
from jax.experimental.pallas import tpu as pltpu
from jax.experimental.pallas import tpu_sc as plsc

N = 10000
E = 320000
G = 64
F_IN = 128
H = 32
N_MICRO = 30

NC = 2
NS = 16
NW = NC * NS
CH = 128
NCH = 80
NBUF = 4
EP = NW * NCH * CH
NP = 10112
RPT = NP // NS

_mesh = plsc.VectorSubcoreMesh(
    core_axis_name="c", subcore_axis_name="s", num_cores=NC, num_subcores=NS)
_sc_params = pltpu.CompilerParams(use_tc_tiling_on_sc=False)


@functools.partial(
    pl.kernel,
    out_type=jax.ShapeDtypeStruct((NC, NP, 16), jnp.float32),
    mesh=_mesh,
    scratch_types=[
        pltpu.VMEM((NCH, CH), jnp.int32),
        pltpu.VMEM((CH, 16), jnp.float32),
        pltpu.VMEM_SHARED((NP, 16), jnp.float32),
    ],
    compiler_params=_sc_params,
)
def _sc_degree(dst_hbm, ones_hbm, z16_hbm, out_hbm, dst_v, ones_v, acc_sh):
    c = lax.axis_index("c")
    s = lax.axis_index("s")
    wid = c * NS + s
    pltpu.sync_copy(z16_hbm.at[pl.ds(s * RPT, RPT)], acc_sh.at[pl.ds(s * RPT, RPT)])
    pltpu.sync_copy(dst_hbm.at[wid], dst_v)
    pltpu.sync_copy(ones_hbm, ones_v)
    plsc.subcore_barrier()

    def body(j, carry):
        pltpu.sync_copy(ones_v, acc_sh.at[dst_v.at[j]], add=True)
        return carry

    lax.fori_loop(0, NCH, body, 0)
    plsc.subcore_barrier()
    pltpu.sync_copy(acc_sh.at[pl.ds(s * RPT, RPT)],
                    out_hbm.at[c, pl.ds(s * RPT, RPT)])


@functools.partial(
    pl.kernel,
    out_type=jax.ShapeDtypeStruct((NC, NP, H), jnp.float32),
    mesh=_mesh,
    scratch_types=[
        pltpu.VMEM((NCH, CH), jnp.int32),
        pltpu.VMEM((NCH, CH), jnp.int32),
        [pltpu.VMEM((CH, H), jnp.float32)] * NBUF,
        [pltpu.SemaphoreType.DMA] * NBUF,
        pltpu.VMEM_SHARED((NP, H), jnp.float32),
    ],
    compiler_params=_sc_params,
)
def _sc_agg(hp_hbm, src_hbm, dst_hbm, z_hbm, out_hbm, src_v, dst_v, rows,
            gsem, acc_sh):
    c = lax.axis_index("c")
    s = lax.axis_index("s")
    wid = c * NS + s
    pltpu.sync_copy(z_hbm.at[pl.ds(s * RPT, RPT)], acc_sh.at[pl.ds(s * RPT, RPT)])
    pltpu.sync_copy(src_hbm.at[wid], src_v)
    pltpu.sync_copy(dst_hbm.at[wid], dst_v)
    plsc.subcore_barrier()

    def start_gather(i, chunk):
        pltpu.async_copy(hp_hbm.at[src_v.at[chunk]], rows[i], gsem[i])

    def wait_gather(i, chunk):
        pltpu.make_async_copy(hp_hbm.at[src_v.at[chunk]], rows[i], gsem[i]).wait()

    for i in range(NBUF):
        start_gather(i, i)

    def body(j, carry):
        base = j * NBUF
        for i in range(NBUF):
            wait_gather(i, base + i)
            pltpu.sync_copy(rows[i], acc_sh.at[dst_v.at[base + i]], add=True)
            start_gather(i, base + NBUF + i)
        return carry

    lax.fori_loop(0, NCH // NBUF - 1, body, 0)
    base = NCH - NBUF
    for i in range(NBUF):
        wait_gather(i, base + i)
        pltpu.sync_copy(rows[i], acc_sh.at[dst_v.at[base + i]], add=True)
    plsc.subcore_barrier()
    pltpu.sync_copy(acc_sh.at[pl.ds(s * RPT, RPT)],
                    out_hbm.at[c, pl.ds(s * RPT, RPT)])


def _tc_pre_body(x_ref, w1_ref, degp_ref, hp_ref, dinv_ref):
    deg = degp_ref[0, :, 0:1] + degp_ref[1, :, 0:1] + 1.0
    dinv = lax.rsqrt(deg)
    h = jnp.dot(x_ref[...], w1_ref[...], preferred_element_type=jnp.float32)
    hp_ref[...] = dinv * h
    dinv_ref[...] = dinv


def _tc_mid_body(aggp_ref, hp_ref, dinv_ref, b_ref, w_ref, out_ref):
    dinv = dinv_ref[...]
    pre = dinv * (aggp_ref[0] + aggp_ref[1] + hp_ref[...]) + b_ref[...]
    a = jnp.maximum(pre, 0.0)
    out_ref[...] = dinv * jnp.dot(a, w_ref[...], preferred_element_type=jnp.float32)


def _tc_head_body(aggp_ref, hp_ref, dinv_ref, b_ref, batch_ref,
                  fw1_ref, fb1_ref, fw2_ref, fb2_ref, out_ref):
    dinv = dinv_ref[...]
    pre = dinv * (aggp_ref[0] + aggp_ref[1] + hp_ref[...]) + b_ref[...]
    a = jnp.maximum(pre, 0.0)[:N, :]
    gid = lax.broadcasted_iota(jnp.int32, (G, N), 0)
    oh = (gid == batch_ref[...]).astype(jnp.float32)
    sums = jnp.dot(oh, a, preferred_element_type=jnp.float32)
    cnt = jnp.sum(oh, axis=1, keepdims=True)
    pooled = sums / jnp.maximum(cnt, 1.0)
    z = jnp.maximum(
        jnp.dot(pooled, fw1_ref[...], preferred_element_type=jnp.float32)
        + fb1_ref[...], 0.0)
    z2 = jnp.dot(z, fw2_ref[...], preferred_element_type=jnp.float32) + fb2_ref[...]
    m = jnp.max(z2, axis=1, keepdims=True)
    e = jnp.exp(z2 - m)
    out_ref[...] = e / jnp.sum(e, axis=1, keepdims=True)


_tc_pre = pl.pallas_call(
    _tc_pre_body,
    out_shape=(jax.ShapeDtypeStruct((NP, H), jnp.float32),
               jax.ShapeDtypeStruct((NP, 1), jnp.float32)),
)

_tc_mid = pl.pallas_call(
    _tc_mid_body,
    out_shape=jax.ShapeDtypeStruct((NP, H), jnp.float32),
)

_tc_head = pl.pallas_call(
    _tc_head_body,
    out_shape=jax.ShapeDtypeStruct((G, N_MICRO), jnp.float32),
)


def kernel(x, edge_index, batch, W1, b1, W2, b2, W3, b3, W4, b4,
           fW1, fb1, fW2, fb2):
    src = edge_index[0]
    dst = edge_index[1]
    pad = EP - E
    src3 = jnp.concatenate([src, jnp.zeros((pad,), jnp.int32)]).reshape(NW, NCH, CH)
    dst3 = jnp.concatenate([dst, jnp.full((pad,), N, jnp.int32)]).reshape(NW, NCH, CH)

    ones16 = jnp.ones((CH, 16), jnp.float32)
    z16 = jnp.zeros((NP, 16), jnp.float32)
    zH = jnp.zeros((NP, H), jnp.float32)
    x_pad = jnp.concatenate([x, jnp.zeros((NP - N, F_IN), x.dtype)], axis=0)
    batch2 = batch.reshape(1, N)

    degp = _sc_degree(dst3, ones16, z16)
    hp, dinv = _tc_pre(x_pad, W1, degp)

    aggp = _sc_agg(hp, src3, dst3, zH)
    hp = _tc_mid(aggp, hp, dinv, b1.reshape(1, H), W2)

    aggp = _sc_agg(hp, src3, dst3, zH)
    hp = _tc_mid(aggp, hp, dinv, b2.reshape(1, H), W3)

    aggp = _sc_agg(hp, src3, dst3, zH)
    hp = _tc_mid(aggp, hp, dinv, b3.reshape(1, H), W4)

    aggp = _sc_agg(hp, src3, dst3, zH)
    return _tc_head(aggp, hp, dinv, b4.reshape(1, H), batch2,
                    fW1, fb1.reshape(1, 64), fW2, fb2.reshape(1, N_MICRO))


# speedup vs baseline: 1.8195x; 1.7747x over previous
import functools

import jax
import jax.numpy as jnp
from jax import lax
from jax.experimental import pallas as pl
from jax.experimental.pallas import tpu as pltpu
from jax.experimental.pallas import tpu_sc as plsc

N = 10000
E = 320000
G = 64
F_IN = 128
H = 32
N_MICRO = 30

NC = 2
NS = 16
NW = NC * NS
CH = 128
NCH = 80
NBUF = 4
EP = NW * NCH * CH
NP = 10112
RPT = NP // NS

_mesh = plsc.VectorSubcoreMesh(
    core_axis_name="c", subcore_axis_name="s", num_cores=NC, num_subcores=NS)
_sc_params = pltpu.CompilerParams(use_tc_tiling_on_sc=False)


@functools.partial(
    pl.kernel,
    out_type=jax.ShapeDtypeStruct((NC, NP, 16), jnp.float32),
    mesh=_mesh,
    scratch_types=[
        pltpu.VMEM((NCH, CH), jnp.int32),
        pltpu.VMEM((CH, 16), jnp.float32),
        pltpu.VMEM_SHARED((NP, 16), jnp.float32),
    ],
    compiler_params=_sc_params,
)
def _sc_degree(dst_hbm, ones_hbm, z16_hbm, out_hbm, dst_v, ones_v, acc_sh):
    c = lax.axis_index("c")
    s = lax.axis_index("s")
    wid = c * NS + s
    pltpu.sync_copy(z16_hbm.at[pl.ds(s * RPT, RPT)], acc_sh.at[pl.ds(s * RPT, RPT)])
    pltpu.sync_copy(dst_hbm.at[wid], dst_v)
    pltpu.sync_copy(ones_hbm, ones_v)
    plsc.subcore_barrier()

    def body(j, carry):
        pltpu.sync_copy(ones_v, acc_sh.at[dst_v.at[j]], add=True)
        return carry

    lax.fori_loop(0, NCH, body, 0)
    plsc.subcore_barrier()
    pltpu.sync_copy(acc_sh.at[pl.ds(s * RPT, RPT)],
                    out_hbm.at[c, pl.ds(s * RPT, RPT)])


@functools.partial(
    pl.kernel,
    out_type=jax.ShapeDtypeStruct((NC, NP, H), jnp.float32),
    mesh=_mesh,
    scratch_types=[
        pltpu.VMEM((NCH, CH), jnp.int32),
        pltpu.VMEM((NCH, CH), jnp.int32),
        [pltpu.VMEM((CH, H), jnp.float32)] * NBUF,
        [pltpu.SemaphoreType.DMA] * NBUF,
        pltpu.VMEM_SHARED((NP, H), jnp.float32),
        pltpu.VMEM_SHARED((NP, H), jnp.float32),
    ],
    compiler_params=_sc_params,
)
def _sc_agg(hp_hbm, src_hbm, dst_hbm, z_hbm, out_hbm, src_v, dst_v, rows,
            gsem, acc_sh, hp_sh):
    c = lax.axis_index("c")
    s = lax.axis_index("s")
    wid = c * NS + s
    pltpu.sync_copy(z_hbm.at[pl.ds(s * RPT, RPT)], acc_sh.at[pl.ds(s * RPT, RPT)])
    pltpu.sync_copy(hp_hbm.at[pl.ds(s * RPT, RPT)], hp_sh.at[pl.ds(s * RPT, RPT)])
    pltpu.sync_copy(src_hbm.at[wid], src_v)
    pltpu.sync_copy(dst_hbm.at[wid], dst_v)
    plsc.subcore_barrier()

    def body(j, carry):
        pltpu.sync_copy(hp_sh.at[src_v.at[j]], rows[0])
        pltpu.sync_copy(rows[0], acc_sh.at[dst_v.at[j]], add=True)
        return carry

    lax.fori_loop(0, NCH, body, 0)
    plsc.subcore_barrier()
    pltpu.sync_copy(acc_sh.at[pl.ds(s * RPT, RPT)],
                    out_hbm.at[c, pl.ds(s * RPT, RPT)])


def _tc_pre_body(x_ref, w1_ref, degp_ref, hp_ref, dinv_ref):
    deg = degp_ref[0, :, 0:1] + degp_ref[1, :, 0:1] + 1.0
    dinv = lax.rsqrt(deg)
    h = jnp.dot(x_ref[...], w1_ref[...], preferred_element_type=jnp.float32)
    hp_ref[...] = dinv * h
    dinv_ref[...] = dinv


def _tc_mid_body(aggp_ref, hp_ref, dinv_ref, b_ref, w_ref, out_ref):
    dinv = dinv_ref[...]
    pre = dinv * (aggp_ref[0] + aggp_ref[1] + hp_ref[...]) + b_ref[...]
    a = jnp.maximum(pre, 0.0)
    out_ref[...] = dinv * jnp.dot(a, w_ref[...], preferred_element_type=jnp.float32)


def _tc_head_body(aggp_ref, hp_ref, dinv_ref, b_ref, batch_ref,
                  fw1_ref, fb1_ref, fw2_ref, fb2_ref, out_ref):
    dinv = dinv_ref[...]
    pre = dinv * (aggp_ref[0] + aggp_ref[1] + hp_ref[...]) + b_ref[...]
    a = jnp.maximum(pre, 0.0)[:N, :]
    gid = lax.broadcasted_iota(jnp.int32, (G, N), 0)
    oh = (gid == batch_ref[...]).astype(jnp.float32)
    sums = jnp.dot(oh, a, preferred_element_type=jnp.float32)
    cnt = jnp.sum(oh, axis=1, keepdims=True)
    pooled = sums / jnp.maximum(cnt, 1.0)
    z = jnp.maximum(
        jnp.dot(pooled, fw1_ref[...], preferred_element_type=jnp.float32)
        + fb1_ref[...], 0.0)
    z2 = jnp.dot(z, fw2_ref[...], preferred_element_type=jnp.float32) + fb2_ref[...]
    m = jnp.max(z2, axis=1, keepdims=True)
    e = jnp.exp(z2 - m)
    out_ref[...] = e / jnp.sum(e, axis=1, keepdims=True)


_tc_pre = pl.pallas_call(
    _tc_pre_body,
    out_shape=(jax.ShapeDtypeStruct((NP, H), jnp.float32),
               jax.ShapeDtypeStruct((NP, 1), jnp.float32)),
)

_tc_mid = pl.pallas_call(
    _tc_mid_body,
    out_shape=jax.ShapeDtypeStruct((NP, H), jnp.float32),
)

_tc_head = pl.pallas_call(
    _tc_head_body,
    out_shape=jax.ShapeDtypeStruct((G, N_MICRO), jnp.float32),
)


def kernel(x, edge_index, batch, W1, b1, W2, b2, W3, b3, W4, b4,
           fW1, fb1, fW2, fb2):
    src = edge_index[0]
    dst = edge_index[1]
    pad = EP - E
    src3 = jnp.concatenate([src, jnp.zeros((pad,), jnp.int32)]).reshape(NW, NCH, CH)
    dst3 = jnp.concatenate([dst, jnp.full((pad,), N, jnp.int32)]).reshape(NW, NCH, CH)

    ones16 = jnp.ones((CH, 16), jnp.float32)
    z16 = jnp.zeros((NP, 16), jnp.float32)
    zH = jnp.zeros((NP, H), jnp.float32)
    x_pad = jnp.concatenate([x, jnp.zeros((NP - N, F_IN), x.dtype)], axis=0)
    batch2 = batch.reshape(1, N)

    degp = _sc_degree(dst3, ones16, z16)
    hp, dinv = _tc_pre(x_pad, W1, degp)

    aggp = _sc_agg(hp, src3, dst3, zH)
    hp = _tc_mid(aggp, hp, dinv, b1.reshape(1, H), W2)

    aggp = _sc_agg(hp, src3, dst3, zH)
    hp = _tc_mid(aggp, hp, dinv, b2.reshape(1, H), W3)

    aggp = _sc_agg(hp, src3, dst3, zH)
    hp = _tc_mid(aggp, hp, dinv, b3.reshape(1, H), W4)

    aggp = _sc_agg(hp, src3, dst3, zH)
    return _tc_head(aggp, hp, dinv, b4.reshape(1, H), batch2,
                    fW1, fb1.reshape(1, 64), fW2, fb2.reshape(1, N_MICRO))


# speedup vs baseline: 1.8208x; 1.0007x over previous
import functools

import jax
import jax.numpy as jnp
from jax import lax
from jax.experimental import pallas as pl
from jax.experimental.pallas import tpu as pltpu
from jax.experimental.pallas import tpu_sc as plsc

N = 10000
E = 320000
G = 64
F_IN = 128
H = 32
N_MICRO = 30

NC = 2
NS = 16
NW = NC * NS
CH = 128
NCH = 80
NBUF = 4
EP = NW * NCH * CH
NP = 10112
RPT = NP // NS

_mesh = plsc.VectorSubcoreMesh(
    core_axis_name="c", subcore_axis_name="s", num_cores=NC, num_subcores=NS)
_sc_params = pltpu.CompilerParams(use_tc_tiling_on_sc=False)


@functools.partial(
    pl.kernel,
    out_type=jax.ShapeDtypeStruct((NC, NP, 16), jnp.float32),
    mesh=_mesh,
    scratch_types=[
        pltpu.VMEM((NCH, CH), jnp.int32),
        pltpu.VMEM((CH, 16), jnp.float32),
        pltpu.VMEM_SHARED((NP, 16), jnp.float32),
    ],
    compiler_params=_sc_params,
)
def _sc_degree(dst_hbm, ones_hbm, z16_hbm, out_hbm, dst_v, ones_v, acc_sh):
    c = lax.axis_index("c")
    s = lax.axis_index("s")
    wid = c * NS + s
    pltpu.sync_copy(z16_hbm.at[pl.ds(s * RPT, RPT)], acc_sh.at[pl.ds(s * RPT, RPT)])
    pltpu.sync_copy(dst_hbm.at[wid], dst_v)
    pltpu.sync_copy(ones_hbm, ones_v)
    plsc.subcore_barrier()

    def body(j, carry):
        pltpu.sync_copy(ones_v, acc_sh.at[dst_v.at[j]], add=True)
        return carry

    lax.fori_loop(0, NCH, body, 0)
    plsc.subcore_barrier()
    pltpu.sync_copy(acc_sh.at[pl.ds(s * RPT, RPT)],
                    out_hbm.at[c, pl.ds(s * RPT, RPT)])


@functools.partial(
    pl.kernel,
    out_type=jax.ShapeDtypeStruct((NC, NP, H), jnp.float32),
    mesh=_mesh,
    scratch_types=[
        pltpu.VMEM((NCH, CH), jnp.int32),
        pltpu.VMEM((NCH, CH), jnp.int32),
        [pltpu.VMEM((CH, H), jnp.float32)] * NBUF,
        [pltpu.SemaphoreType.DMA] * NBUF,
        pltpu.VMEM_SHARED((NP, H), jnp.float32),
        pltpu.VMEM_SHARED((NP, H), jnp.float32),
    ],
    compiler_params=_sc_params,
)
def _sc_agg(hp_hbm, src_hbm, dst_hbm, z_hbm, out_hbm, src_v, dst_v, rows,
            gsem, acc_sh, hp_sh):
    c = lax.axis_index("c")
    s = lax.axis_index("s")
    wid = c * NS + s
    pltpu.sync_copy(z_hbm.at[pl.ds(s * RPT, RPT)], acc_sh.at[pl.ds(s * RPT, RPT)])
    pltpu.sync_copy(hp_hbm.at[pl.ds(s * RPT, RPT)], hp_sh.at[pl.ds(s * RPT, RPT)])
    pltpu.sync_copy(src_hbm.at[wid], src_v)
    pltpu.sync_copy(dst_hbm.at[wid], dst_v)
    plsc.subcore_barrier()

    def body(j, carry):
        pltpu.sync_copy(hp_sh.at[src_v.at[j]], rows[0])
        pltpu.sync_copy(rows[0], acc_sh.at[dst_v.at[j]], add=True)
        return carry

    lax.fori_loop(0, NCH, body, 0)
    plsc.subcore_barrier()
    pltpu.sync_copy(acc_sh.at[pl.ds(s * RPT, RPT)],
                    out_hbm.at[c, pl.ds(s * RPT, RPT)])


def _tc_mm_body(x_ref, w1_ref, h_ref):
    h_ref[...] = jnp.dot(x_ref[...], w1_ref[...],
                         preferred_element_type=jnp.float32)


def _tc_scale_body(h_ref, degp_ref, hp_ref, dinv_ref):
    deg = degp_ref[0, :, 0:1] + degp_ref[1, :, 0:1] + 1.0
    dinv = lax.rsqrt(deg)
    hp_ref[...] = dinv * h_ref[...]
    dinv_ref[...] = dinv


def _tc_mid_body(aggp_ref, hp_ref, dinv_ref, b_ref, w_ref, out_ref):
    dinv = dinv_ref[...]
    pre = dinv * (aggp_ref[0] + aggp_ref[1] + hp_ref[...]) + b_ref[...]
    a = jnp.maximum(pre, 0.0)
    out_ref[...] = dinv * jnp.dot(a, w_ref[...], preferred_element_type=jnp.float32)


def _tc_head_body(aggp_ref, hp_ref, dinv_ref, b_ref, batch_ref,
                  fw1_ref, fb1_ref, fw2_ref, fb2_ref, out_ref):
    dinv = dinv_ref[...]
    pre = dinv * (aggp_ref[0] + aggp_ref[1] + hp_ref[...]) + b_ref[...]
    a = jnp.maximum(pre, 0.0)[:N, :]
    gid = lax.broadcasted_iota(jnp.int32, (G, N), 0)
    oh = (gid == batch_ref[...]).astype(jnp.float32)
    sums = jnp.dot(oh, a, preferred_element_type=jnp.float32)
    cnt = jnp.sum(oh, axis=1, keepdims=True)
    pooled = sums / jnp.maximum(cnt, 1.0)
    z = jnp.maximum(
        jnp.dot(pooled, fw1_ref[...], preferred_element_type=jnp.float32)
        + fb1_ref[...], 0.0)
    z2 = jnp.dot(z, fw2_ref[...], preferred_element_type=jnp.float32) + fb2_ref[...]
    m = jnp.max(z2, axis=1, keepdims=True)
    e = jnp.exp(z2 - m)
    out_ref[...] = e / jnp.sum(e, axis=1, keepdims=True)


_tc_mm = pl.pallas_call(
    _tc_mm_body,
    out_shape=jax.ShapeDtypeStruct((NP, H), jnp.float32),
)

_tc_scale = pl.pallas_call(
    _tc_scale_body,
    out_shape=(jax.ShapeDtypeStruct((NP, H), jnp.float32),
               jax.ShapeDtypeStruct((NP, 1), jnp.float32)),
)

_tc_mid = pl.pallas_call(
    _tc_mid_body,
    out_shape=jax.ShapeDtypeStruct((NP, H), jnp.float32),
)

_tc_head = pl.pallas_call(
    _tc_head_body,
    out_shape=jax.ShapeDtypeStruct((G, N_MICRO), jnp.float32),
)


def kernel(x, edge_index, batch, W1, b1, W2, b2, W3, b3, W4, b4,
           fW1, fb1, fW2, fb2):
    src = edge_index[0]
    dst = edge_index[1]
    pad = EP - E
    src3 = jnp.concatenate([src, jnp.zeros((pad,), jnp.int32)]).reshape(NW, NCH, CH)
    dst3 = jnp.concatenate([dst, jnp.full((pad,), N, jnp.int32)]).reshape(NW, NCH, CH)

    ones16 = jnp.ones((CH, 16), jnp.float32)
    z16 = jnp.zeros((NP, 16), jnp.float32)
    zH = jnp.zeros((NP, H), jnp.float32)
    x_pad = jnp.concatenate([x, jnp.zeros((NP - N, F_IN), x.dtype)], axis=0)
    batch2 = batch.reshape(1, N)

    h1 = _tc_mm(x_pad, W1)
    degp = _sc_degree(dst3, ones16, z16)
    hp, dinv = _tc_scale(h1, degp)

    aggp = _sc_agg(hp, src3, dst3, zH)
    hp = _tc_mid(aggp, hp, dinv, b1.reshape(1, H), W2)

    aggp = _sc_agg(hp, src3, dst3, zH)
    hp = _tc_mid(aggp, hp, dinv, b2.reshape(1, H), W3)

    aggp = _sc_agg(hp, src3, dst3, zH)
    hp = _tc_mid(aggp, hp, dinv, b3.reshape(1, H), W4)

    aggp = _sc_agg(hp, src3, dst3, zH)
    return _tc_head(aggp, hp, dinv, b4.reshape(1, H), batch2,
                    fW1, fb1.reshape(1, 64), fW2, fb2.reshape(1, N_MICRO))


# speedup vs baseline: 2.3366x; 1.2833x over previous
import functools

import jax
import jax.numpy as jnp
from jax import lax
from jax.experimental import pallas as pl
from jax.experimental.pallas import tpu as pltpu
from jax.experimental.pallas import tpu_sc as plsc

N = 10000
E = 320000
G = 64
F_IN = 128
H = 32
N_MICRO = 30

NC = 2
NS = 16
NW = NC * NS
CH = 128
NCH = 80
NBUF = 4
EP = NW * NCH * CH
NP = 10112
RPT = NP // NS
NPP = NP // 4
RPP = RPT // 4
NQ = N // 4

_mesh = plsc.VectorSubcoreMesh(
    core_axis_name="c", subcore_axis_name="s", num_cores=NC, num_subcores=NS)
_sc_params = pltpu.CompilerParams(use_tc_tiling_on_sc=False)


@functools.partial(
    pl.kernel,
    out_type=jax.ShapeDtypeStruct((NC, NPP, 128), jnp.float32),
    mesh=_mesh,
    scratch_types=[
        pltpu.VMEM((NCH, CH), jnp.int32),
        pltpu.VMEM((CH, 16), jnp.float32),
        pltpu.VMEM((RPT, 16), jnp.float32),
        pltpu.VMEM((RPP, 128), jnp.float32),
        pltpu.VMEM_SHARED((NP, 16), jnp.float32),
    ],
    compiler_params=_sc_params,
)
def _sc_degree(dst_hbm, ones_hbm, z16_hbm, out_hbm, dst_v, ones_v, cnt_v,
               pk_v, acc_sh):
    c = lax.axis_index("c")
    s = lax.axis_index("s")
    wid = c * NS + s
    pltpu.sync_copy(z16_hbm.at[pl.ds(s * RPT, RPT)], acc_sh.at[pl.ds(s * RPT, RPT)])
    pltpu.sync_copy(dst_hbm.at[wid], dst_v)
    pltpu.sync_copy(ones_hbm, ones_v)
    plsc.subcore_barrier()

    def body(j, carry):
        pltpu.sync_copy(ones_v, acc_sh.at[dst_v.at[j]], add=True)
        return carry

    lax.fori_loop(0, NCH, body, 0)
    plsc.subcore_barrier()
    pltpu.sync_copy(acc_sh.at[pl.ds(s * RPT, RPT)], cnt_v)

    def repack(i, carry):
        for a in range(4):
            v = cnt_v[4 * i + a]
            pk_v[i, pl.ds(32 * a, 16)] = v
            pk_v[i, pl.ds(32 * a + 16, 16)] = v
        return carry

    lax.fori_loop(0, RPP, repack, 0)
    pltpu.sync_copy(pk_v, out_hbm.at[c, pl.ds(s * RPP, RPP)])


@functools.partial(
    pl.kernel,
    out_type=jax.ShapeDtypeStruct((NC, NP, H), jnp.float32),
    mesh=_mesh,
    scratch_types=[
        pltpu.VMEM((NCH, CH), jnp.int32),
        pltpu.VMEM((NCH, CH), jnp.int32),
        [pltpu.VMEM((CH, H), jnp.float32)] * NBUF,
        [pltpu.SemaphoreType.DMA] * NBUF,
        pltpu.VMEM_SHARED((NP, H), jnp.float32),
        pltpu.VMEM_SHARED((NP, H), jnp.float32),
    ],
    compiler_params=_sc_params,
)
def _sc_agg(hp_hbm, src_hbm, dst_hbm, z_hbm, out_hbm, src_v, dst_v, rows,
            gsem, acc_sh, hp_sh):
    c = lax.axis_index("c")
    s = lax.axis_index("s")
    wid = c * NS + s
    pltpu.sync_copy(z_hbm.at[pl.ds(s * RPT, RPT)], acc_sh.at[pl.ds(s * RPT, RPT)])
    pltpu.sync_copy(hp_hbm.at[pl.ds(s * RPT, RPT)], hp_sh.at[pl.ds(s * RPT, RPT)])
    pltpu.sync_copy(src_hbm.at[wid], src_v)
    pltpu.sync_copy(dst_hbm.at[wid], dst_v)
    plsc.subcore_barrier()

    def body(j, carry):
        pltpu.sync_copy(hp_sh.at[src_v.at[j]], rows[0])
        pltpu.sync_copy(rows[0], acc_sh.at[dst_v.at[j]], add=True)
        return carry

    lax.fori_loop(0, NCH, body, 0)
    plsc.subcore_barrier()
    pltpu.sync_copy(acc_sh.at[pl.ds(s * RPT, RPT)],
                    out_hbm.at[c, pl.ds(s * RPT, RPT)])


def _tc_mm_body(x_ref, w1_ref, h_ref):
    h_ref[...] = jnp.dot(x_ref[...], w1_ref[...],
                         preferred_element_type=jnp.float32)


def _tc_scale_body(h_ref, degp_ref, hp_ref, dinv_ref):
    dinv = lax.rsqrt(degp_ref[0] + degp_ref[1] + 1.0)
    hp_ref[...] = dinv * h_ref[...]
    dinv_ref[...] = dinv


def _tc_mid_body(aggp_ref, hp_ref, dinv_ref, b_ref, w_ref, out_ref):
    dinv = dinv_ref[...]
    pre = dinv * (aggp_ref[0] + aggp_ref[1] + hp_ref[...]) + b_ref[...]
    a = jnp.maximum(pre, 0.0)
    out_ref[...] = dinv * jnp.dot(a, w_ref[...], preferred_element_type=jnp.float32)


def _tc_head_body(aggp_ref, hp_ref, dinv_ref, b_ref, b0_ref, b1_ref, b2_ref,
                  b3_ref, fw1_ref, fb1_ref, fw2_ref, fb2_ref, out_ref):
    dinv = dinv_ref[...]
    pre = dinv * (aggp_ref[0] + aggp_ref[1] + hp_ref[...]) + b_ref[...]
    a = jnp.maximum(pre, 0.0)[:NQ, :]
    gid = lax.broadcasted_iota(jnp.int32, (G, NQ), 0)
    pooled = jnp.zeros((G, H), jnp.float32)
    cnt = jnp.zeros((G, 1), jnp.float32)
    for ai, bref in enumerate((b0_ref, b1_ref, b2_ref, b3_ref)):
        oh = (gid == bref[...]).astype(jnp.float32)
        res = jnp.dot(oh, a, preferred_element_type=jnp.float32)
        pooled = pooled + res[:, 32 * ai:32 * ai + 32]
        cnt = cnt + jnp.sum(oh, axis=1, keepdims=True)
    pooled = pooled / jnp.maximum(cnt, 1.0)
    z = jnp.maximum(
        jnp.dot(pooled, fw1_ref[...], preferred_element_type=jnp.float32)
        + fb1_ref[...], 0.0)
    z2 = jnp.dot(z, fw2_ref[...], preferred_element_type=jnp.float32) + fb2_ref[...]
    m = jnp.max(z2, axis=1, keepdims=True)
    e = jnp.exp(z2 - m)
    out_ref[...] = e / jnp.sum(e, axis=1, keepdims=True)


_tc_mm = pl.pallas_call(
    _tc_mm_body,
    out_shape=jax.ShapeDtypeStruct((NPP, 128), jnp.float32),
)

_tc_scale = pl.pallas_call(
    _tc_scale_body,
    out_shape=(jax.ShapeDtypeStruct((NPP, 128), jnp.float32),
               jax.ShapeDtypeStruct((NPP, 128), jnp.float32)),
)

_tc_mid = pl.pallas_call(
    _tc_mid_body,
    out_shape=jax.ShapeDtypeStruct((NPP, 128), jnp.float32),
)

_tc_head = pl.pallas_call(
    _tc_head_body,
    out_shape=jax.ShapeDtypeStruct((G, N_MICRO), jnp.float32),
)


def kernel(x, edge_index, batch, W1, b1, W2, b2, W3, b3, W4, b4,
           fW1, fb1, fW2, fb2):
    src = edge_index[0]
    dst = edge_index[1]
    pad = EP - E
    src3 = jnp.concatenate([src, jnp.zeros((pad,), jnp.int32)]).reshape(NW, NCH, CH)
    dst3 = jnp.concatenate([dst, jnp.full((pad,), N, jnp.int32)]).reshape(NW, NCH, CH)

    ones16 = jnp.ones((CH, 16), jnp.float32)
    z16 = jnp.zeros((NP, 16), jnp.float32)
    zH = jnp.zeros((NP, H), jnp.float32)
    xp = jnp.concatenate([x, jnp.zeros((NP - N, F_IN), x.dtype)]).reshape(NPP, 512)
    eye4 = jnp.eye(4, dtype=jnp.float32)
    wb1 = jnp.kron(eye4, W1)
    wb2 = jnp.kron(eye4, W2)
    wb3 = jnp.kron(eye4, W3)
    wb4 = jnp.kron(eye4, W4)
    bt1 = jnp.tile(b1, 4).reshape(1, 128)
    bt2 = jnp.tile(b2, 4).reshape(1, 128)
    bt3 = jnp.tile(b3, 4).reshape(1, 128)
    bt4 = jnp.tile(b4, 4).reshape(1, 128)
    batch_a = [batch[a::4].reshape(1, NQ) for a in range(4)]

    h1 = _tc_mm(xp, wb1)
    degp = _sc_degree(dst3, ones16, z16)
    hp, dinv = _tc_scale(h1, degp)

    aggp = _sc_agg(hp.reshape(NP, H), src3, dst3, zH)
    hp = _tc_mid(aggp.reshape(NC, NPP, 128), hp, dinv, bt1, wb2)

    aggp = _sc_agg(hp.reshape(NP, H), src3, dst3, zH)
    hp = _tc_mid(aggp.reshape(NC, NPP, 128), hp, dinv, bt2, wb3)

    aggp = _sc_agg(hp.reshape(NP, H), src3, dst3, zH)
    hp = _tc_mid(aggp.reshape(NC, NPP, 128), hp, dinv, bt3, wb4)

    aggp = _sc_agg(hp.reshape(NP, H), src3, dst3, zH)
    return _tc_head(aggp.reshape(NC, NPP, 128), hp, dinv, bt4,
                    batch_a[0], batch_a[1], batch_a[2], batch_a[3],
                    fW1, fb1.reshape(1, 64), fW2, fb2.reshape(1, N_MICRO))


# speedup vs baseline: 2.4104x; 1.0315x over previous
import functools

import jax
import jax.numpy as jnp
from jax import lax
from jax.experimental import pallas as pl
from jax.experimental.pallas import tpu as pltpu
from jax.experimental.pallas import tpu_sc as plsc

N = 10000
E = 320000
G = 64
F_IN = 128
H = 32
N_MICRO = 30

NC = 2
NS = 16
NW = NC * NS
CH = 128
EQ = E // CH
NCHF = EQ // NW
NREM = EQ - NCHF * NW
NCH = NCHF + 1
NBUF = 4
NP = 10112
RPT = NP // NS
NPP = NP // 4
RPP = RPT // 4
NQ = N // 4

_mesh = plsc.VectorSubcoreMesh(
    core_axis_name="c", subcore_axis_name="s", num_cores=NC, num_subcores=NS)
_sc_params = pltpu.CompilerParams(use_tc_tiling_on_sc=False)


@functools.partial(
    pl.kernel,
    out_type=jax.ShapeDtypeStruct((NC, NPP, 128), jnp.float32),
    mesh=_mesh,
    scratch_types=[
        pltpu.VMEM((NCH, CH), jnp.int32),
        pltpu.VMEM((CH, 16), jnp.float32),
        pltpu.VMEM((RPT, 16), jnp.float32),
        pltpu.VMEM((RPP, 128), jnp.float32),
        pltpu.VMEM_SHARED((NP, 16), jnp.float32),
    ],
    compiler_params=_sc_params,
)
def _sc_degree(dst_hbm, ones_hbm, z16_hbm, out_hbm, dst_v, ones_v, cnt_v,
               pk_v, acc_sh):
    c = lax.axis_index("c")
    s = lax.axis_index("s")
    wid = c * NS + s
    row0 = NCHF * wid + jnp.maximum(0, wid - (NW - NREM))
    nch = jnp.where(wid >= NW - NREM, NCHF + 1, NCHF)
    pltpu.sync_copy(z16_hbm.at[pl.ds(s * RPT, RPT)], acc_sh.at[pl.ds(s * RPT, RPT)])
    pltpu.sync_copy(dst_hbm.at[pl.ds(row0, NCH)], dst_v)
    pltpu.sync_copy(ones_hbm, ones_v)
    plsc.subcore_barrier()

    def body(j, carry):
        pltpu.sync_copy(ones_v, acc_sh.at[dst_v.at[j]], add=True)
        return carry

    lax.fori_loop(0, nch, body, 0)
    plsc.subcore_barrier()
    pltpu.sync_copy(acc_sh.at[pl.ds(s * RPT, RPT)], cnt_v)

    def repack(i, carry):
        for a in range(4):
            v = cnt_v[4 * i + a]
            pk_v[i, pl.ds(32 * a, 16)] = v
            pk_v[i, pl.ds(32 * a + 16, 16)] = v
        return carry

    lax.fori_loop(0, RPP, repack, 0)
    pltpu.sync_copy(pk_v, out_hbm.at[c, pl.ds(s * RPP, RPP)])


@functools.partial(
    pl.kernel,
    out_type=jax.ShapeDtypeStruct((NC, NP, H), jnp.float32),
    mesh=_mesh,
    scratch_types=[
        pltpu.VMEM((NCH, CH), jnp.int32),
        pltpu.VMEM((NCH, CH), jnp.int32),
        [pltpu.VMEM((CH, H), jnp.float32)] * NBUF,
        [pltpu.SemaphoreType.DMA] * NBUF,
        pltpu.VMEM_SHARED((NP, H), jnp.float32),
        pltpu.VMEM_SHARED((NP, H), jnp.float32),
    ],
    compiler_params=_sc_params,
)
def _sc_agg(hp_hbm, src_hbm, dst_hbm, z_hbm, out_hbm, src_v, dst_v, rows,
            gsem, acc_sh, hp_sh):
    c = lax.axis_index("c")
    s = lax.axis_index("s")
    wid = c * NS + s
    row0 = NCHF * wid + jnp.maximum(0, wid - (NW - NREM))
    nch = jnp.where(wid >= NW - NREM, NCHF + 1, NCHF)
    pltpu.sync_copy(z_hbm.at[pl.ds(s * RPT, RPT)], acc_sh.at[pl.ds(s * RPT, RPT)])
    pltpu.sync_copy(hp_hbm.at[pl.ds(s * RPT, RPT)], hp_sh.at[pl.ds(s * RPT, RPT)])
    pltpu.sync_copy(src_hbm.at[pl.ds(row0, NCH)], src_v)
    pltpu.sync_copy(dst_hbm.at[pl.ds(row0, NCH)], dst_v)
    plsc.subcore_barrier()

    def body(j, carry):
        pltpu.sync_copy(hp_sh.at[src_v.at[j]], rows[0])
        pltpu.sync_copy(rows[0], acc_sh.at[dst_v.at[j]], add=True)
        return carry

    lax.fori_loop(0, nch, body, 0)
    plsc.subcore_barrier()
    pltpu.sync_copy(acc_sh.at[pl.ds(s * RPT, RPT)],
                    out_hbm.at[c, pl.ds(s * RPT, RPT)])


def _tc_mm_body(x_ref, w1_ref, h_ref):
    h = jnp.dot(x_ref[...], w1_ref[...], preferred_element_type=jnp.float32)
    h_ref[...] = jnp.concatenate(
        [h, jnp.zeros((NPP - NQ, 128), jnp.float32)], axis=0)


def _tc_scale_body(h_ref, degp_ref, hp_ref, dinv_ref):
    dinv = lax.rsqrt(degp_ref[0] + degp_ref[1] + 1.0)
    hp_ref[...] = dinv * h_ref[...]
    dinv_ref[...] = dinv


def _tc_mid_body(aggp_ref, hp_ref, dinv_ref, b_ref, w_ref, out_ref):
    dinv = dinv_ref[...]
    pre = dinv * (aggp_ref[0] + aggp_ref[1] + hp_ref[...]) + b_ref[...]
    a = jnp.maximum(pre, 0.0)
    out_ref[...] = dinv * jnp.dot(a, w_ref[...], preferred_element_type=jnp.float32)


def _tc_head_body(aggp_ref, hp_ref, dinv_ref, b_ref, b0_ref, b1_ref, b2_ref,
                  b3_ref, fw1_ref, fb1_ref, fw2_ref, fb2_ref, out_ref):
    dinv = dinv_ref[...]
    pre = dinv * (aggp_ref[0] + aggp_ref[1] + hp_ref[...]) + b_ref[...]
    a = jnp.maximum(pre, 0.0)[:NQ, :]
    gid = lax.broadcasted_iota(jnp.int32, (G, NQ), 0)
    pooled = jnp.zeros((G, H), jnp.float32)
    cnt = jnp.zeros((G, 1), jnp.float32)
    for ai, bref in enumerate((b0_ref, b1_ref, b2_ref, b3_ref)):
        oh = (gid == bref[...]).astype(jnp.float32)
        res = jnp.dot(oh, a, preferred_element_type=jnp.float32)
        pooled = pooled + res[:, 32 * ai:32 * ai + 32]
        cnt = cnt + jnp.sum(oh, axis=1, keepdims=True)
    pooled = pooled / jnp.maximum(cnt, 1.0)
    z = jnp.maximum(
        jnp.dot(pooled, fw1_ref[...], preferred_element_type=jnp.float32)
        + fb1_ref[...], 0.0)
    z2 = jnp.dot(z, fw2_ref[...], preferred_element_type=jnp.float32) + fb2_ref[...]
    m = jnp.max(z2, axis=1, keepdims=True)
    e = jnp.exp(z2 - m)
    out_ref[...] = e / jnp.sum(e, axis=1, keepdims=True)


_tc_mm = pl.pallas_call(
    _tc_mm_body,
    out_shape=jax.ShapeDtypeStruct((NPP, 128), jnp.float32),
)

_tc_scale = pl.pallas_call(
    _tc_scale_body,
    out_shape=(jax.ShapeDtypeStruct((NPP, 128), jnp.float32),
               jax.ShapeDtypeStruct((NPP, 128), jnp.float32)),
)

_tc_mid = pl.pallas_call(
    _tc_mid_body,
    out_shape=jax.ShapeDtypeStruct((NPP, 128), jnp.float32),
)

_tc_head = pl.pallas_call(
    _tc_head_body,
    out_shape=jax.ShapeDtypeStruct((G, N_MICRO), jnp.float32),
)


def kernel(x, edge_index, batch, W1, b1, W2, b2, W3, b3, W4, b4,
           fW1, fb1, fW2, fb2):
    src2 = edge_index[0].reshape(EQ, CH)
    dst2 = edge_index[1].reshape(EQ, CH)

    ones16 = jnp.ones((CH, 16), jnp.float32)
    z16 = jnp.zeros((NP, 16), jnp.float32)
    zH = jnp.zeros((NP, H), jnp.float32)
    xp = x.reshape(NQ, 512)
    eye4 = jnp.eye(4, dtype=jnp.float32)
    wb1 = jnp.kron(eye4, W1)
    wb2 = jnp.kron(eye4, W2)
    wb3 = jnp.kron(eye4, W3)
    wb4 = jnp.kron(eye4, W4)
    bt1 = jnp.tile(b1, 4).reshape(1, 128)
    bt2 = jnp.tile(b2, 4).reshape(1, 128)
    bt3 = jnp.tile(b3, 4).reshape(1, 128)
    bt4 = jnp.tile(b4, 4).reshape(1, 128)
    batch_a = [batch[a::4].reshape(1, NQ) for a in range(4)]

    h1 = _tc_mm(xp, wb1)
    degp = _sc_degree(dst2, ones16, z16)
    hp, dinv = _tc_scale(h1, degp)

    aggp = _sc_agg(hp.reshape(NP, H), src2, dst2, zH)
    hp = _tc_mid(aggp.reshape(NC, NPP, 128), hp, dinv, bt1, wb2)

    aggp = _sc_agg(hp.reshape(NP, H), src2, dst2, zH)
    hp = _tc_mid(aggp.reshape(NC, NPP, 128), hp, dinv, bt2, wb3)

    aggp = _sc_agg(hp.reshape(NP, H), src2, dst2, zH)
    hp = _tc_mid(aggp.reshape(NC, NPP, 128), hp, dinv, bt3, wb4)

    aggp = _sc_agg(hp.reshape(NP, H), src2, dst2, zH)
    return _tc_head(aggp.reshape(NC, NPP, 128), hp, dinv, bt4,
                    batch_a[0], batch_a[1], batch_a[2], batch_a[3],
                    fW1, fb1.reshape(1, 64), fW2, fb2.reshape(1, N_MICRO))


# speedup vs baseline: 2.4184x; 1.0033x over previous
import functools

import jax
import jax.numpy as jnp
from jax import lax
from jax.experimental import pallas as pl
from jax.experimental.pallas import tpu as pltpu
from jax.experimental.pallas import tpu_sc as plsc

N = 10000
E = 320000
G = 64
F_IN = 128
H = 32
N_MICRO = 30

NC = 2
NS = 16
NW = NC * NS
CH = 128
EQ = E // CH
NCHF = EQ // NW
NREM = EQ - NCHF * NW
NCH = NCHF + 1
NBUF = 4
NP = 10112
RPT = NP // NS
NPP = NP // 4
RPP = RPT // 4
NQ = N // 4

_mesh = plsc.VectorSubcoreMesh(
    core_axis_name="c", subcore_axis_name="s", num_cores=NC, num_subcores=NS)
_sc_params = pltpu.CompilerParams(use_tc_tiling_on_sc=False)


@functools.partial(
    pl.kernel,
    out_type=jax.ShapeDtypeStruct((NC, NPP, 128), jnp.float32),
    mesh=_mesh,
    scratch_types=[
        pltpu.VMEM((NCH, CH), jnp.int32),
        pltpu.VMEM((CH, 16), jnp.float32),
        pltpu.VMEM((RPT, 16), jnp.float32),
        pltpu.VMEM((RPP, 128), jnp.float32),
        pltpu.VMEM_SHARED((NP, 16), jnp.float32),
    ],
    compiler_params=_sc_params,
)
def _sc_degree(dst_hbm, ones_hbm, z16_hbm, out_hbm, dst_v, ones_v, cnt_v,
               pk_v, acc_sh):
    c = lax.axis_index("c")
    s = lax.axis_index("s")
    wid = c * NS + s
    row0 = NCHF * wid + jnp.maximum(0, wid - (NW - NREM))
    nch = jnp.where(wid >= NW - NREM, NCHF + 1, NCHF)
    pltpu.sync_copy(z16_hbm.at[pl.ds(s * RPT, RPT)], acc_sh.at[pl.ds(s * RPT, RPT)])
    pltpu.sync_copy(dst_hbm.at[pl.ds(row0, NCH)], dst_v)
    pltpu.sync_copy(ones_hbm, ones_v)
    plsc.subcore_barrier()

    def body(j, carry):
        pltpu.sync_copy(ones_v, acc_sh.at[dst_v.at[j]], add=True)
        return carry

    lax.fori_loop(0, nch, body, 0)
    plsc.subcore_barrier()
    pltpu.sync_copy(acc_sh.at[pl.ds(s * RPT, RPT)], cnt_v)

    def repack(i, carry):
        for a in range(4):
            v = cnt_v[4 * i + a]
            pk_v[i, pl.ds(32 * a, 16)] = v
            pk_v[i, pl.ds(32 * a + 16, 16)] = v
        return carry

    lax.fori_loop(0, RPP, repack, 0)
    pltpu.sync_copy(pk_v, out_hbm.at[c, pl.ds(s * RPP, RPP)])


@functools.partial(
    pl.kernel,
    out_type=jax.ShapeDtypeStruct((NC, NP, H), jnp.float32),
    mesh=_mesh,
    scratch_types=[
        pltpu.VMEM((NCH, CH), jnp.int32),
        pltpu.VMEM((NCH, CH), jnp.int32),
        pltpu.VMEM((CH, H), jnp.float32),
        pltpu.VMEM_SHARED((NP, H), jnp.float32),
        pltpu.VMEM_SHARED((NP, H), jnp.float32),
    ],
    compiler_params=_sc_params,
)
def _sc_agg(hp_hbm, src_hbm, dst_hbm, z_hbm, out_hbm, src_v, dst_v, rows,
            acc_sh, hp_sh):
    c = lax.axis_index("c")
    s = lax.axis_index("s")
    wid = c * NS + s
    row0 = NCHF * wid + jnp.maximum(0, wid - (NW - NREM))
    nch = jnp.where(wid >= NW - NREM, NCHF + 1, NCHF)
    pltpu.sync_copy(z_hbm.at[pl.ds(s * RPT, RPT)], acc_sh.at[pl.ds(s * RPT, RPT)])
    pltpu.sync_copy(hp_hbm.at[pl.ds(s * RPT, RPT)], hp_sh.at[pl.ds(s * RPT, RPT)])
    pltpu.sync_copy(src_hbm.at[pl.ds(row0, NCH)], src_v)
    pltpu.sync_copy(dst_hbm.at[pl.ds(row0, NCH)], dst_v)
    plsc.subcore_barrier()

    def body(j, carry):
        pltpu.sync_copy(hp_sh.at[src_v.at[j]], rows)
        pltpu.sync_copy(rows, acc_sh.at[dst_v.at[j]], add=True)
        return carry

    lax.fori_loop(0, nch, body, 0)
    plsc.subcore_barrier()
    pltpu.sync_copy(acc_sh.at[pl.ds(s * RPT, RPT)],
                    out_hbm.at[c, pl.ds(s * RPT, RPT)])


def _tc_mm_body(x_ref, w1_ref, h_ref):
    h = jnp.dot(x_ref[...], w1_ref[...], preferred_element_type=jnp.float32)
    h_ref[...] = jnp.concatenate(
        [h, jnp.zeros((NPP - NQ, 128), jnp.float32)], axis=0)


def _tc_scale_body(h_ref, degp_ref, hp_ref, dinv_ref):
    dinv = lax.rsqrt(degp_ref[0] + degp_ref[1] + 1.0)
    hp_ref[...] = dinv * h_ref[...]
    dinv_ref[...] = dinv


def _tc_mid_body(aggp_ref, hp_ref, dinv_ref, b_ref, w_ref, out_ref):
    dinv = dinv_ref[...]
    pre = dinv * (aggp_ref[0] + aggp_ref[1] + hp_ref[...]) + b_ref[...]
    a = jnp.maximum(pre, 0.0)
    out_ref[...] = dinv * jnp.dot(a, w_ref[...], preferred_element_type=jnp.float32)


def _tc_head_body(aggp_ref, hp_ref, dinv_ref, b_ref, bt_ref,
                  fw1_ref, fb1_ref, fw2_ref, fb2_ref, out_ref):
    dinv = dinv_ref[...]
    pre = dinv * (aggp_ref[0] + aggp_ref[1] + hp_ref[...]) + b_ref[...]
    a = jnp.maximum(pre, 0.0)[:NQ, :]
    gid = lax.broadcasted_iota(jnp.int32, (G, NQ), 0)
    pooled = jnp.zeros((G, H), jnp.float32)
    cnt = jnp.zeros((G, 1), jnp.float32)
    for ai in range(4):
        oh = (gid == bt_ref[ai:ai + 1, :]).astype(jnp.float32)
        res = jnp.dot(oh, a, preferred_element_type=jnp.float32)
        pooled = pooled + res[:, 32 * ai:32 * ai + 32]
        cnt = cnt + jnp.sum(oh, axis=1, keepdims=True)
    pooled = pooled / jnp.maximum(cnt, 1.0)
    z = jnp.maximum(
        jnp.dot(pooled, fw1_ref[...], preferred_element_type=jnp.float32)
        + fb1_ref[...], 0.0)
    z2 = jnp.dot(z, fw2_ref[...], preferred_element_type=jnp.float32) + fb2_ref[...]
    m = jnp.max(z2, axis=1, keepdims=True)
    e = jnp.exp(z2 - m)
    out_ref[...] = e / jnp.sum(e, axis=1, keepdims=True)


_tc_mm = pl.pallas_call(
    _tc_mm_body,
    out_shape=jax.ShapeDtypeStruct((NPP, 128), jnp.float32),
)

_tc_scale = pl.pallas_call(
    _tc_scale_body,
    out_shape=(jax.ShapeDtypeStruct((NPP, 128), jnp.float32),
               jax.ShapeDtypeStruct((NPP, 128), jnp.float32)),
)

_tc_mid = pl.pallas_call(
    _tc_mid_body,
    out_shape=jax.ShapeDtypeStruct((NPP, 128), jnp.float32),
)

_tc_head = pl.pallas_call(
    _tc_head_body,
    out_shape=jax.ShapeDtypeStruct((G, N_MICRO), jnp.float32),
)


def kernel(x, edge_index, batch, W1, b1, W2, b2, W3, b3, W4, b4,
           fW1, fb1, fW2, fb2):
    src2 = edge_index[0].reshape(EQ, CH)
    dst2 = edge_index[1].reshape(EQ, CH)

    ones16 = jnp.ones((CH, 16), jnp.float32)
    z16 = jnp.zeros((NP, 16), jnp.float32)
    zH = jnp.zeros((NP, H), jnp.float32)
    xp = x.reshape(NQ, 512)
    eye4 = jnp.eye(4, dtype=jnp.float32)
    wb1 = jnp.kron(eye4, W1)
    wb2 = jnp.kron(eye4, W2)
    wb3 = jnp.kron(eye4, W3)
    wb4 = jnp.kron(eye4, W4)
    bt1 = jnp.tile(b1, 4).reshape(1, 128)
    bt2 = jnp.tile(b2, 4).reshape(1, 128)
    bt3 = jnp.tile(b3, 4).reshape(1, 128)
    bt4 = jnp.tile(b4, 4).reshape(1, 128)
    batch_t = batch.reshape(NQ, 4).T

    h1 = _tc_mm(xp, wb1)
    degp = _sc_degree(dst2, ones16, z16)
    hp, dinv = _tc_scale(h1, degp)

    aggp = _sc_agg(hp.reshape(NP, H), src2, dst2, zH)
    hp = _tc_mid(aggp.reshape(NC, NPP, 128), hp, dinv, bt1, wb2)

    aggp = _sc_agg(hp.reshape(NP, H), src2, dst2, zH)
    hp = _tc_mid(aggp.reshape(NC, NPP, 128), hp, dinv, bt2, wb3)

    aggp = _sc_agg(hp.reshape(NP, H), src2, dst2, zH)
    hp = _tc_mid(aggp.reshape(NC, NPP, 128), hp, dinv, bt3, wb4)

    aggp = _sc_agg(hp.reshape(NP, H), src2, dst2, zH)
    return _tc_head(aggp.reshape(NC, NPP, 128), hp, dinv, bt4, batch_t,
                    fW1, fb1.reshape(1, 64), fW2, fb2.reshape(1, N_MICRO))
